# baseline (speedup 1.0000x reference)
import math

import jax
import jax.numpy as jnp
from jax.experimental import pallas as pl
from jax.experimental.pallas import tpu as pltpu

TEMPERATURE = 0.1
BLOCK_Q = 2048
NCHUNK = 4


def _attn_body(z_q_ref, wq_ref, bq_ref, wk_ref, bk_ref, wv_ref, bv_ref,
               out_ref, k_scr, v_scr):
    i = pl.program_id(0)
    nproj = pl.num_programs(0) // 2

    @pl.when(i < nproj)
    def _project_kv():
        zf = z_q_ref[...].astype(jnp.bfloat16)
        off = i * BLOCK_Q
        k = jax.lax.dot_general(
            zf, wk_ref[...], (((1,), (1,)), ((), ())),
            preferred_element_type=jnp.float32) + bk_ref[...]
        k_scr[pl.ds(off, BLOCK_Q), :] = k.astype(jnp.bfloat16)
        v = jax.lax.dot_general(
            zf, wv_ref[...], (((1,), (1,)), ((), ())),
            preferred_element_type=jnp.float32) + bv_ref[...]
        v_scr[pl.ds(off, BLOCK_Q), :] = v.astype(jnp.bfloat16)

    @pl.when(i >= nproj)
    def _attend():
        h = BLOCK_Q // NCHUNK
        for hb in range(NCHUNK):
            sl = pl.ds(hb * h, h)
            z_q = z_q_ref[sl, :]
            q = jax.lax.dot_general(
                z_q.astype(jnp.bfloat16), wq_ref[...], (((1,), (1,)), ((), ())),
                preferred_element_type=jnp.float32) + bq_ref[...]
            s = jax.lax.dot_general(
                q.astype(jnp.bfloat16), k_scr[...], (((1,), (1,)), ((), ())),
                preferred_element_type=jnp.float32)
            m = jnp.max(s, axis=1, keepdims=True)
            p = jnp.exp2(s - m)
            denom = jnp.sum(p, axis=1, keepdims=True)
            o = jax.lax.dot_general(
                p.astype(jnp.bfloat16), v_scr[...], (((1,), (0,)), ((), ())),
                preferred_element_type=jnp.float32)
            out_ref[sl, :] = z_q + o * (0.5 / denom)


def kernel(z, labels, Wq, bq, Wk, bk, Wv, bv, memory, memory_weights,
           memory_labels, running_mean, running_cov):
    B, d = z.shape
    c = math.log2(math.e) / (math.sqrt(d) * TEMPERATURE)
    wq16 = (Wq * c).astype(jnp.bfloat16)
    bq_s = (bq * c).reshape(1, d)
    wk16 = Wk.astype(jnp.bfloat16)
    wv16 = Wv.astype(jnp.bfloat16)
    bk2 = bk.reshape(1, d)
    bv2 = bv.reshape(1, d)
    nb = B // BLOCK_Q
    zq_idx = lambda i: (jnp.where(i < nb, i, i - nb), 0)
    full = lambda i: (0, 0)
    out = pl.pallas_call(
        _attn_body,
        grid=(2 * nb,),
        in_specs=[
            pl.BlockSpec((BLOCK_Q, d), zq_idx),
            pl.BlockSpec((d, d), full),
            pl.BlockSpec((1, d), full),
            pl.BlockSpec((d, d), full),
            pl.BlockSpec((1, d), full),
            pl.BlockSpec((d, d), full),
            pl.BlockSpec((1, d), full),
        ],
        out_specs=pl.BlockSpec((BLOCK_Q, d), zq_idx),
        out_shape=jax.ShapeDtypeStruct((B, d), jnp.float32),
        scratch_shapes=[
            pltpu.VMEM((B, d), jnp.bfloat16),
            pltpu.VMEM((B, d), jnp.bfloat16),
        ],
    )(z, wq16, bq_s, wk16, bk2, wv16, bv2)
    return out


# speedup vs baseline: 1.1249x; 1.1249x over previous
import math

import jax
import jax.numpy as jnp
from jax.experimental import pallas as pl
from jax.experimental.pallas import tpu as pltpu

TEMPERATURE = 0.1
NCHUNK = 8


def _attn_body(z_ref, wq_ref, bq_ref, wk_ref, bk_ref, wv_ref, bv_ref,
               out_ref, k_scr, v_scr):
    zf = z_ref[...].astype(jnp.bfloat16)
    k = jax.lax.dot_general(
        zf, wk_ref[...], (((1,), (1,)), ((), ())),
        preferred_element_type=jnp.float32) + bk_ref[...]
    k_scr[...] = k.astype(jnp.bfloat16)
    v = jax.lax.dot_general(
        zf, wv_ref[...], (((1,), (1,)), ((), ())),
        preferred_element_type=jnp.float32) + bv_ref[...]
    v_scr[...] = v.astype(jnp.bfloat16)

    B = z_ref.shape[0]
    h = B // NCHUNK
    for hb in range(NCHUNK):
        sl = pl.ds(hb * h, h)
        z_q = z_ref[sl, :]
        q = jax.lax.dot_general(
            z_q.astype(jnp.bfloat16), wq_ref[...], (((1,), (1,)), ((), ())),
            preferred_element_type=jnp.float32) + bq_ref[...]
        s = jax.lax.dot_general(
            q.astype(jnp.bfloat16), k_scr[...], (((1,), (1,)), ((), ())),
            preferred_element_type=jnp.float32)
        m = jnp.max(s, axis=1, keepdims=True)
        p = jnp.exp2(s - m)
        denom = jnp.sum(p, axis=1, keepdims=True)
        o = jax.lax.dot_general(
            p.astype(jnp.bfloat16), v_scr[...], (((1,), (0,)), ((), ())),
            preferred_element_type=jnp.float32)
        out_ref[sl, :] = z_q + o * (0.5 / denom)


def kernel(z, labels, Wq, bq, Wk, bk, Wv, bv, memory, memory_weights,
           memory_labels, running_mean, running_cov):
    B, d = z.shape
    c = math.log2(math.e) / (math.sqrt(d) * TEMPERATURE)
    wq16 = (Wq * c).astype(jnp.bfloat16)
    bq_s = (bq * c).reshape(1, d)
    wk16 = Wk.astype(jnp.bfloat16)
    wv16 = Wv.astype(jnp.bfloat16)
    bk2 = bk.reshape(1, d)
    bv2 = bv.reshape(1, d)
    out = pl.pallas_call(
        _attn_body,
        out_shape=jax.ShapeDtypeStruct((B, d), jnp.float32),
        scratch_shapes=[
            pltpu.VMEM((B, d), jnp.bfloat16),
            pltpu.VMEM((B, d), jnp.bfloat16),
        ],
    )(z, wq16, bq_s, wk16, bk2, wv16, bv2)
    return out


# speedup vs baseline: 1.4295x; 1.2708x over previous
import math

import jax
import jax.numpy as jnp
from jax.experimental import pallas as pl
from jax.experimental.pallas import tpu as pltpu

TEMPERATURE = 0.1
NCHUNK = 8


def _attn_body(z_ref, wq_ref, bq_ref, wk_ref, bk_ref, wv_ref, bv_ref,
               out_ref, k_scr, v_scr):
    B, d = z_ref.shape
    c = math.log2(math.e) / (math.sqrt(d) * TEMPERATURE)
    zf = z_ref[...].astype(jnp.bfloat16)
    k = jax.lax.dot_general(
        zf, wk_ref[...].astype(jnp.bfloat16), (((1,), (1,)), ((), ())),
        preferred_element_type=jnp.float32) + bk_ref[...]
    k_scr[...] = k.astype(jnp.bfloat16)
    v = jax.lax.dot_general(
        zf, wv_ref[...].astype(jnp.bfloat16), (((1,), (1,)), ((), ())),
        preferred_element_type=jnp.float32) + bv_ref[...]
    v_scr[:, :d] = v.astype(jnp.bfloat16)
    v_scr[:, d:] = jnp.ones((B, d), jnp.bfloat16)
    wq16 = (wq_ref[...] * c).astype(jnp.bfloat16)
    bq_s = bq_ref[...] * c

    h = B // NCHUNK
    for hb in range(NCHUNK):
        sl = pl.ds(hb * h, h)
        z_q = z_ref[sl, :]
        q = jax.lax.dot_general(
            z_q.astype(jnp.bfloat16), wq16, (((1,), (1,)), ((), ())),
            preferred_element_type=jnp.float32) + bq_s
        s = jax.lax.dot_general(
            q.astype(jnp.bfloat16), k_scr[...], (((1,), (1,)), ((), ())),
            preferred_element_type=jnp.float32).astype(jnp.bfloat16)
        m = jnp.max(s, axis=1, keepdims=True)
        p = jnp.exp2(s - m)
        o_cat = jax.lax.dot_general(
            p, v_scr[...], (((1,), (0,)), ((), ())),
            preferred_element_type=jnp.float32)
        out_ref[sl, :] = z_q + o_cat[:, :d] * (0.5 / o_cat[:, d:])


def kernel(z, labels, Wq, bq, Wk, bk, Wv, bv, memory, memory_weights,
           memory_labels, running_mean, running_cov):
    B, d = z.shape
    out = pl.pallas_call(
        _attn_body,
        out_shape=jax.ShapeDtypeStruct((B, d), jnp.float32),
        scratch_shapes=[
            pltpu.VMEM((B, d), jnp.bfloat16),
            pltpu.VMEM((B, 2 * d), jnp.bfloat16),
        ],
    )(z, Wq, bq.reshape(1, d), Wk, bk.reshape(1, d), Wv, bv.reshape(1, d))
    return out


# speedup vs baseline: 1.4912x; 1.0432x over previous
import math

import jax
import jax.numpy as jnp
from jax.experimental import pallas as pl
from jax.experimental.pallas import tpu as pltpu

TEMPERATURE = 0.1
NCHUNK = 8


def _attn_body(z_ref, wq_ref, bq_ref, wk_ref, bk_ref, wv_ref, bv_ref,
               out_ref, k_scr, v_scr, q_scr):
    B, d = z_ref.shape
    c = math.log2(math.e) / (math.sqrt(d) * TEMPERATURE)
    zf = z_ref[...].astype(jnp.bfloat16)
    k = jax.lax.dot_general(
        zf, wk_ref[...].astype(jnp.bfloat16), (((1,), (1,)), ((), ())),
        preferred_element_type=jnp.float32) + bk_ref[...]
    k_scr[...] = k.astype(jnp.bfloat16)
    v = jax.lax.dot_general(
        zf, wv_ref[...].astype(jnp.bfloat16), (((1,), (1,)), ((), ())),
        preferred_element_type=jnp.float32) + bv_ref[...]
    v_scr[:, :d] = v.astype(jnp.bfloat16)
    v_scr[:, d:] = jnp.ones((B, d), jnp.bfloat16)
    wq16 = (wq_ref[...] * c).astype(jnp.bfloat16)
    bq_s = bq_ref[...] * c
    q_all = jax.lax.dot_general(
        zf, wq16, (((1,), (1,)), ((), ())),
        preferred_element_type=jnp.float32) + bq_s
    q_scr[...] = q_all.astype(jnp.bfloat16)

    h = B // NCHUNK
    for hb in range(NCHUNK):
        sl = pl.ds(hb * h, h)
        z_q = z_ref[sl, :]
        s = jax.lax.dot_general(
            q_scr[sl, :], k_scr[...], (((1,), (1,)), ((), ())),
            preferred_element_type=jnp.float32).astype(jnp.bfloat16)
        m = jnp.max(s, axis=1, keepdims=True)
        p = jnp.exp2(s - m)
        o_cat = jax.lax.dot_general(
            p, v_scr[...], (((1,), (0,)), ((), ())),
            preferred_element_type=jnp.float32)
        out_ref[sl, :] = z_q + o_cat[:, :d] * (0.5 / o_cat[:, d:])


def kernel(z, labels, Wq, bq, Wk, bk, Wv, bv, memory, memory_weights,
           memory_labels, running_mean, running_cov):
    B, d = z.shape
    out = pl.pallas_call(
        _attn_body,
        out_shape=jax.ShapeDtypeStruct((B, d), jnp.float32),
        scratch_shapes=[
            pltpu.VMEM((B, d), jnp.bfloat16),
            pltpu.VMEM((B, 2 * d), jnp.bfloat16),
            pltpu.VMEM((B, d), jnp.bfloat16),
        ],
    )(z, Wq, bq.reshape(1, d), Wk, bk.reshape(1, d), Wv, bv.reshape(1, d))
    return out


# speedup vs baseline: 1.6127x; 1.0815x over previous
import math

import jax
import jax.numpy as jnp
from jax.experimental import pallas as pl
from jax.experimental.pallas import tpu as pltpu

TEMPERATURE = 0.1
NCHUNK = 32


def _attn_body(z_ref, wq_ref, bq_ref, wk_ref, bk_ref, wv_ref, bv_ref,
               out_ref, k_scr, v_scr, q_scr):
    B, d = z_ref.shape
    c = math.log2(math.e) / (math.sqrt(d) * TEMPERATURE)
    zf = z_ref[...].astype(jnp.bfloat16)
    k = jax.lax.dot_general(
        zf, wk_ref[...].astype(jnp.bfloat16), (((1,), (1,)), ((), ())),
        preferred_element_type=jnp.float32) + bk_ref[...]
    k_scr[...] = k.astype(jnp.bfloat16)
    v = jax.lax.dot_general(
        zf, wv_ref[...].astype(jnp.bfloat16), (((1,), (1,)), ((), ())),
        preferred_element_type=jnp.float32) + bv_ref[...]
    v_scr[:, :d] = v.astype(jnp.bfloat16)
    v_scr[:, d:] = jnp.ones((B, d), jnp.bfloat16)
    wq16 = (wq_ref[...] * c).astype(jnp.bfloat16)
    bq_s = bq_ref[...] * c
    q_all = jax.lax.dot_general(
        zf, wq16, (((1,), (1,)), ((), ())),
        preferred_element_type=jnp.float32) + bq_s
    q_scr[...] = q_all.astype(jnp.bfloat16)

    h = B // NCHUNK
    for hb in range(NCHUNK):
        sl = pl.ds(hb * h, h)
        z_q = z_ref[sl, :]
        s = jax.lax.dot_general(
            q_scr[sl, :], k_scr[...], (((1,), (1,)), ((), ())),
            preferred_element_type=jnp.float32).astype(jnp.bfloat16)
        m = jnp.max(s, axis=1, keepdims=True)
        p = jnp.exp2(s - m)
        o_cat = jax.lax.dot_general(
            p, v_scr[...], (((1,), (0,)), ((), ())),
            preferred_element_type=jnp.float32)
        out_ref[sl, :] = z_q + o_cat[:, :d] * (0.5 / o_cat[:, d:])


def kernel(z, labels, Wq, bq, Wk, bk, Wv, bv, memory, memory_weights,
           memory_labels, running_mean, running_cov):
    B, d = z.shape
    out = pl.pallas_call(
        _attn_body,
        out_shape=jax.ShapeDtypeStruct((B, d), jnp.float32),
        scratch_shapes=[
            pltpu.VMEM((B, d), jnp.bfloat16),
            pltpu.VMEM((B, 2 * d), jnp.bfloat16),
            pltpu.VMEM((B, d), jnp.bfloat16),
        ],
    )(z, Wq, bq.reshape(1, d), Wk, bk.reshape(1, d), Wv, bv.reshape(1, d))
    return out
